# Initial kernel scaffold; baseline (speedup 1.0000x reference)
import jax
import jax.numpy as jnp
from jax.experimental import pallas as pl


def kernel(edge_index, edge_attrs, table):
    raise NotImplementedError("write your pallas kernel here")


# speedup vs baseline: 6.5719x; 6.5719x over previous
import functools

import jax
import jax.numpy as jnp
from jax import lax
from jax.experimental import pallas as pl
from jax.experimental.pallas import tpu as pltpu
from jax.experimental.pallas import tpu_sc as plsc

N_NODES = 50000
EMB_DIM = 64
N_EDGES = 800000
N_LAYERS = 3

NC = 2
NS = 16
LANES = 16

HALF = N_NODES // NC
ACC_ROWS = 25088
ROWS_PER_TILE = ACC_ROWS // NS
DUMMY = HALF

B = 128
CHUNKS = -(-N_EDGES // (NS * B))
E_PAD = NS * B * CHUNKS
EDGES_PER_TILE = B * CHUNKS

CH = 64
Z_CH = 32
SCALE_CHUNKS = -(-HALF // (NS * CH))
LAST_START = HALF - CH

_MESH = plsc.VectorSubcoreMesh(core_axis_name="c", subcore_axis_name="s")


def _lane_bcast(vec, lane):
    idx = jnp.full((LANES,), lane, jnp.int32)
    return jnp.take_along_axis(vec, idx, axis=0)


def _rsqrt16(d):
    i = lax.bitcast_convert_type(d, jnp.int32)
    i = 0x5F3759DF - jnp.right_shift(i, 1)
    y = lax.bitcast_convert_type(i, jnp.float32)
    for _ in range(3):
        y = y * (1.5 - 0.5 * d * y * y)
    return jnp.where(d >= 0.5, y, 0.0)


def _local_idx(to_v, idx_v, base):
    for g in range(B // LANES):
        t = to_v[pl.ds(g * LANES, LANES)]
        local = t - base
        ok = jnp.logical_and(local >= 0, local < HALF)
        idx_v[pl.ds(g * LANES, LANES)] = jnp.where(ok, local, DUMMY)


def _scale_chunk_start(j, s):
    g = j * NS + s
    return lax.min(g * CH, LAST_START)


def _deg_kernel_body(to_h, table_h, dinv_h, y0_h,
                     to_v, idx_v, ones_v, zbuf, dchunk, dinvbuf, tbuf, ybuf,
                     deg_s, sem):
    del sem
    c = lax.axis_index("c")
    s = lax.axis_index("s")
    base = c * HALF

    zeros16 = jnp.zeros((LANES,), jnp.float32)
    ones16 = jnp.ones((LANES,), jnp.float32)

    def zfill(i, _):
        zbuf[pl.ds(i * LANES, LANES)] = zeros16
        return 0

    lax.fori_loop(0, ROWS_PER_TILE // LANES, zfill, 0)
    for g in range(B // LANES):
        ones_v[pl.ds(g * LANES, LANES)] = ones16
    pltpu.sync_copy(zbuf, deg_s.at[pl.ds(s * ROWS_PER_TILE, ROWS_PER_TILE)])
    plsc.subcore_barrier()

    def edge_step(j, _):
        ebase = (s * CHUNKS + j) * B
        pltpu.sync_copy(to_h.at[pl.ds(ebase, B)], to_v)
        _local_idx(to_v, idx_v, base)
        pltpu.sync_copy(ones_v, deg_s.at[idx_v], add=True)
        return 0

    lax.fori_loop(0, CHUNKS, edge_step, 0)
    plsc.subcore_barrier()

    def scale_step(j, _):
        start_l = _scale_chunk_start(j, s)
        gstart = base + start_l
        pltpu.sync_copy(deg_s.at[pl.ds(start_l, CH)], dchunk)
        for g in range(CH // LANES):
            d = dchunk[pl.ds(g * LANES, LANES)]
            dinvbuf[pl.ds(g * LANES, LANES)] = _rsqrt16(d)
        pltpu.sync_copy(dinvbuf, dinv_h.at[pl.ds(gstart, CH)])
        pltpu.sync_copy(table_h.at[pl.ds(gstart, CH)], tbuf)

        def group(g, _):
            dvec = dinvbuf[pl.ds(g * LANES, LANES)]
            for r16 in range(LANES):
                d = _lane_bcast(dvec, r16)
                r = g * LANES + r16
                for k in range(EMB_DIM // LANES):
                    sl = pl.ds(k * LANES, LANES)
                    ybuf[r, sl] = tbuf[r, sl] * d
            return 0

        lax.fori_loop(0, CH // LANES, group, 0)
        pltpu.sync_copy(ybuf, y0_h.at[pl.ds(gstart, CH)])
        return 0

    lax.fori_loop(0, SCALE_CHUNKS, scale_step, 0)


def _layer_kernel_body(last, from_h, to_h, y_h, s_h, dinv_h, *refs):
    if last:
        (s_out_h, from_v, to_v, idx_v, rows_v, zrow,
         abuf, sbuf, obuf, dinvbuf, acc_s, sem) = refs
        y_out_h = None
        ybuf = None
    else:
        (s_out_h, y_out_h, from_v, to_v, idx_v, rows_v, zrow,
         abuf, sbuf, obuf, ybuf, dinvbuf, acc_s, sem) = refs

    c = lax.axis_index("c")
    s = lax.axis_index("s")
    base = c * HALF

    zeros16 = jnp.zeros((LANES,), jnp.float32)

    def zfill(r, _):
        for k in range(EMB_DIM // LANES):
            zrow[r, pl.ds(k * LANES, LANES)] = zeros16
        return 0

    lax.fori_loop(0, Z_CH, zfill, 0)

    def zcopy(j, _):
        pltpu.sync_copy(zrow,
                        acc_s.at[pl.ds(s * ROWS_PER_TILE + j * Z_CH, Z_CH)])
        return 0

    lax.fori_loop(0, ROWS_PER_TILE // Z_CH, zcopy, 0)
    plsc.subcore_barrier()

    def edge_step(j, _):
        ebase = (s * CHUNKS + j) * B
        pltpu.sync_copy(from_h.at[pl.ds(ebase, B)], from_v)
        pltpu.sync_copy(to_h.at[pl.ds(ebase, B)], to_v)
        pltpu.async_copy(y_h.at[from_v], rows_v, sem).wait()
        _local_idx(to_v, idx_v, base)
        pltpu.sync_copy(rows_v, acc_s.at[idx_v], add=True)
        return 0

    lax.fori_loop(0, CHUNKS, edge_step, 0)
    plsc.subcore_barrier()

    def scale_step(j, _):
        start_l = _scale_chunk_start(j, s)
        gstart = base + start_l
        pltpu.sync_copy(acc_s.at[pl.ds(start_l, CH)], abuf)
        pltpu.sync_copy(s_h.at[pl.ds(gstart, CH)], sbuf)
        pltpu.sync_copy(dinv_h.at[pl.ds(gstart, CH)], dinvbuf)

        def group(g, _):
            dvec = dinvbuf[pl.ds(g * LANES, LANES)]
            for r16 in range(LANES):
                d = _lane_bcast(dvec, r16)
                r = g * LANES + r16
                for k in range(EMB_DIM // LANES):
                    sl = pl.ds(k * LANES, LANES)
                    e = abuf[r, sl] * d
                    sv = sbuf[r, sl] + e
                    if last:
                        sv = sv * 0.25
                    obuf[r, sl] = sv
                    if not last:
                        ybuf[r, sl] = e * d
            return 0

        lax.fori_loop(0, CH // LANES, group, 0)
        pltpu.sync_copy(obuf, s_out_h.at[pl.ds(gstart, CH)])
        if not last:
            pltpu.sync_copy(ybuf, y_out_h.at[pl.ds(gstart, CH)])
        return 0

    lax.fori_loop(0, SCALE_CHUNKS, scale_step, 0)


_deg_kernel = pl.kernel(
    _deg_kernel_body,
    out_type=[
        jax.ShapeDtypeStruct((N_NODES,), jnp.float32),
        jax.ShapeDtypeStruct((N_NODES, EMB_DIM), jnp.float32),
    ],
    mesh=_MESH,
    compiler_params=pltpu.CompilerParams(use_tc_tiling_on_sc=False),
    scratch_types=[
        pltpu.VMEM((B,), jnp.int32),
        pltpu.VMEM((B,), jnp.int32),
        pltpu.VMEM((B,), jnp.float32),
        pltpu.VMEM((ROWS_PER_TILE,), jnp.float32),
        pltpu.VMEM((CH,), jnp.float32),
        pltpu.VMEM((CH,), jnp.float32),
        pltpu.VMEM((CH, EMB_DIM), jnp.float32),
        pltpu.VMEM((CH, EMB_DIM), jnp.float32),
        pltpu.VMEM_SHARED((ACC_ROWS,), jnp.float32),
        pltpu.SemaphoreType.DMA,
    ],
)


def _layer_out_type(last):
    out = [jax.ShapeDtypeStruct((N_NODES, EMB_DIM), jnp.float32)]
    if not last:
        out.append(jax.ShapeDtypeStruct((N_NODES, EMB_DIM), jnp.float32))
    return out


def _layer_scratch(last):
    scratch = [
        pltpu.VMEM((B,), jnp.int32),
        pltpu.VMEM((B,), jnp.int32),
        pltpu.VMEM((B,), jnp.int32),
        pltpu.VMEM((B, EMB_DIM), jnp.float32),
        pltpu.VMEM((Z_CH, EMB_DIM), jnp.float32),
        pltpu.VMEM((CH, EMB_DIM), jnp.float32),
        pltpu.VMEM((CH, EMB_DIM), jnp.float32),
        pltpu.VMEM((CH, EMB_DIM), jnp.float32),
    ]
    if not last:
        scratch.append(pltpu.VMEM((CH, EMB_DIM), jnp.float32))
    scratch += [
        pltpu.VMEM((CH,), jnp.float32),
        pltpu.VMEM_SHARED((ACC_ROWS, EMB_DIM), jnp.float32),
        pltpu.SemaphoreType.DMA,
    ]
    return scratch


_mid_layer = pl.kernel(
    functools.partial(_layer_kernel_body, False),
    out_type=_layer_out_type(False),
    mesh=_MESH,
    compiler_params=pltpu.CompilerParams(use_tc_tiling_on_sc=False),
    scratch_types=_layer_scratch(False),
)

_last_layer = pl.kernel(
    functools.partial(_layer_kernel_body, True),
    out_type=_layer_out_type(True),
    mesh=_MESH,
    compiler_params=pltpu.CompilerParams(use_tc_tiling_on_sc=False),
    scratch_types=_layer_scratch(True),
)


def kernel(edge_index, edge_attrs, table):
    del edge_attrs
    pad = E_PAD - N_EDGES
    from_p = jnp.concatenate(
        [edge_index[0], jnp.zeros((pad,), jnp.int32)])
    to_p = jnp.concatenate(
        [edge_index[1], jnp.full((pad,), N_NODES, jnp.int32)])

    dinv, y = _deg_kernel(to_p, table)
    s = table
    for layer in range(N_LAYERS):
        if layer == N_LAYERS - 1:
            (s,) = _last_layer(from_p, to_p, y, s, dinv)
        else:
            s, y = _mid_layer(from_p, to_p, y, s, dinv)
    return (table, s)


# speedup vs baseline: 8.4774x; 1.2899x over previous
import functools

import jax
import jax.numpy as jnp
from jax import lax
from jax.experimental import pallas as pl
from jax.experimental.pallas import tpu as pltpu
from jax.experimental.pallas import tpu_sc as plsc

N_NODES = 50000
EMB_DIM = 64
N_EDGES = 800000
N_LAYERS = 3

NC = 2
NS = 16
LANES = 16

HALF = N_NODES // NC
ACC_ROWS = 25088
ROWS_PER_TILE = ACC_ROWS // NS
DUMMY = HALF

B = 128
CHUNKS = -(-N_EDGES // (NS * B))
CHUNKS += CHUNKS % 2
E_PAD = NS * B * CHUNKS
EDGES_PER_TILE = B * CHUNKS

CH = 32
Z_CH = 32
SCALE_CHUNKS = -(-HALF // (NS * CH))
LAST_START = HALF - CH

_MESH = plsc.VectorSubcoreMesh(core_axis_name="c", subcore_axis_name="s")


def _lane_bcast(vec, lane):
    idx = jnp.full((LANES,), lane, jnp.int32)
    return jnp.take_along_axis(vec, idx, axis=0)


def _rsqrt16(d):
    i = lax.bitcast_convert_type(d, jnp.int32)
    i = 0x5F3759DF - jnp.right_shift(i, 1)
    y = lax.bitcast_convert_type(i, jnp.float32)
    for _ in range(3):
        y = y * (1.5 - 0.5 * d * y * y)
    return jnp.where(d >= 0.5, y, 0.0)


def _local_idx(to_v, idx_v, base):
    for g in range(B // LANES):
        t = to_v[pl.ds(g * LANES, LANES)]
        local = t - base
        ok = jnp.logical_and(local >= 0, local < HALF)
        idx_v[pl.ds(g * LANES, LANES)] = jnp.where(ok, local, DUMMY)


def _scale_chunk_start(j, s):
    g = j * NS + s
    return lax.min(g * CH, LAST_START)


def _deg_kernel_body(to_h, table_h, dinv_h, y0_h,
                     to_v, idx_v, ones_v, zbuf, dchunk, dinvbuf, tbuf, ybuf,
                     deg_s, sem):
    del sem
    c = lax.axis_index("c")
    s = lax.axis_index("s")
    base = c * HALF

    zeros16 = jnp.zeros((LANES,), jnp.float32)
    ones16 = jnp.ones((LANES,), jnp.float32)

    def zfill(i, _):
        zbuf[pl.ds(i * LANES, LANES)] = zeros16
        return 0

    lax.fori_loop(0, ROWS_PER_TILE // LANES, zfill, 0)
    for g in range(B // LANES):
        ones_v[pl.ds(g * LANES, LANES)] = ones16
    pltpu.sync_copy(zbuf, deg_s.at[pl.ds(s * ROWS_PER_TILE, ROWS_PER_TILE)])
    plsc.subcore_barrier()

    def edge_step(j, _):
        ebase = (s * CHUNKS + j) * B
        pltpu.sync_copy(to_h.at[pl.ds(ebase, B)], to_v)
        _local_idx(to_v, idx_v, base)
        pltpu.sync_copy(ones_v, deg_s.at[idx_v], add=True)
        return 0

    lax.fori_loop(0, CHUNKS, edge_step, 0)
    plsc.subcore_barrier()

    def scale_step(j, _):
        start_l = _scale_chunk_start(j, s)
        gstart = base + start_l
        pltpu.sync_copy(deg_s.at[pl.ds(start_l, CH)], dchunk)
        for g in range(CH // LANES):
            d = dchunk[pl.ds(g * LANES, LANES)]
            dinvbuf[pl.ds(g * LANES, LANES)] = _rsqrt16(d)
        pltpu.sync_copy(dinvbuf, dinv_h.at[pl.ds(gstart, CH)])
        pltpu.sync_copy(table_h.at[pl.ds(gstart, CH)], tbuf)

        def group(g, _):
            dvec = dinvbuf[pl.ds(g * LANES, LANES)]
            for r16 in range(LANES):
                d = _lane_bcast(dvec, r16)
                r = g * LANES + r16
                for k in range(EMB_DIM // LANES):
                    sl = pl.ds(k * LANES, LANES)
                    ybuf[r, sl] = tbuf[r, sl] * d
            return 0

        lax.fori_loop(0, CH // LANES, group, 0)
        pltpu.sync_copy(ybuf, y0_h.at[pl.ds(gstart, CH)])
        return 0

    lax.fori_loop(0, SCALE_CHUNKS, scale_step, 0)


def _layer_kernel_body(last, from_h, to_h, y_h, s_h, dinv_h, *refs):
    if last:
        (s_out_h, from_v0, from_v1, to_v0, to_v1, idx_v0, idx_v1,
         rows_v0, rows_v1, zrow,
         abuf, sbuf, obuf, dinvbuf, acc_s, gsem0, gsem1) = refs
        y_out_h = None
        ybuf = None
    else:
        (s_out_h, y_out_h, from_v0, from_v1, to_v0, to_v1, idx_v0, idx_v1,
         rows_v0, rows_v1, zrow,
         abuf, sbuf, obuf, ybuf, dinvbuf, acc_s, gsem0, gsem1) = refs
    slots_from = (from_v0, from_v1)
    slots_to = (to_v0, to_v1)
    slots_idx = (idx_v0, idx_v1)
    slots_rows = (rows_v0, rows_v1)
    slots_gsem = (gsem0, gsem1)

    c = lax.axis_index("c")
    s = lax.axis_index("s")
    base = c * HALF

    zeros16 = jnp.zeros((LANES,), jnp.float32)

    def zfill(r, _):
        for k in range(EMB_DIM // LANES):
            zrow[r, pl.ds(k * LANES, LANES)] = zeros16
        return 0

    lax.fori_loop(0, Z_CH, zfill, 0)

    def zcopy(j, _):
        pltpu.sync_copy(zrow,
                        acc_s.at[pl.ds(s * ROWS_PER_TILE + j * Z_CH, Z_CH)])
        return 0

    lax.fori_loop(0, ROWS_PER_TILE // Z_CH, zcopy, 0)
    plsc.subcore_barrier()

    def prep(j, slot):
        ebase = (s * CHUNKS + j) * B
        pltpu.sync_copy(from_h.at[pl.ds(ebase, B)], slots_from[slot])
        pltpu.sync_copy(to_h.at[pl.ds(ebase, B)], slots_to[slot])
        _local_idx(slots_to[slot], slots_idx[slot], base)
        pltpu.async_copy(
            y_h.at[slots_from[slot]], slots_rows[slot], slots_gsem[slot])

    prep(0, 0)

    def outer(i, _):
        for b in (0, 1):
            j = i * 2 + b
            nslot = 1 - b
            if b == 0:
                prep(j + 1, nslot)
            else:
                @pl.when(j + 1 < CHUNKS)
                def _():
                    prep(j + 1, nslot)
            pltpu.make_async_copy(
                y_h.at[slots_from[b]], slots_rows[b], slots_gsem[b]).wait()
            pltpu.sync_copy(slots_rows[b], acc_s.at[slots_idx[b]], add=True)
        return 0

    lax.fori_loop(0, CHUNKS // 2, outer, 0)
    plsc.subcore_barrier()

    def scale_step(j, _):
        start_l = _scale_chunk_start(j, s)
        gstart = base + start_l
        pltpu.sync_copy(acc_s.at[pl.ds(start_l, CH)], abuf)
        pltpu.sync_copy(s_h.at[pl.ds(gstart, CH)], sbuf)
        pltpu.sync_copy(dinv_h.at[pl.ds(gstart, CH)], dinvbuf)

        def group(g, _):
            dvec = dinvbuf[pl.ds(g * LANES, LANES)]
            for r16 in range(LANES):
                d = _lane_bcast(dvec, r16)
                r = g * LANES + r16
                for k in range(EMB_DIM // LANES):
                    sl = pl.ds(k * LANES, LANES)
                    e = abuf[r, sl] * d
                    sv = sbuf[r, sl] + e
                    if last:
                        sv = sv * 0.25
                    obuf[r, sl] = sv
                    if not last:
                        ybuf[r, sl] = e * d
            return 0

        lax.fori_loop(0, CH // LANES, group, 0)
        pltpu.sync_copy(obuf, s_out_h.at[pl.ds(gstart, CH)])
        if not last:
            pltpu.sync_copy(ybuf, y_out_h.at[pl.ds(gstart, CH)])
        return 0

    lax.fori_loop(0, SCALE_CHUNKS, scale_step, 0)


_deg_kernel = pl.kernel(
    _deg_kernel_body,
    out_type=[
        jax.ShapeDtypeStruct((N_NODES,), jnp.float32),
        jax.ShapeDtypeStruct((N_NODES, EMB_DIM), jnp.float32),
    ],
    mesh=_MESH,
    compiler_params=pltpu.CompilerParams(use_tc_tiling_on_sc=False),
    scratch_types=[
        pltpu.VMEM((B,), jnp.int32),
        pltpu.VMEM((B,), jnp.int32),
        pltpu.VMEM((B,), jnp.float32),
        pltpu.VMEM((ROWS_PER_TILE,), jnp.float32),
        pltpu.VMEM((CH,), jnp.float32),
        pltpu.VMEM((CH,), jnp.float32),
        pltpu.VMEM((CH, EMB_DIM), jnp.float32),
        pltpu.VMEM((CH, EMB_DIM), jnp.float32),
        pltpu.VMEM_SHARED((ACC_ROWS,), jnp.float32),
        pltpu.SemaphoreType.DMA,
    ],
)


def _layer_out_type(last):
    out = [jax.ShapeDtypeStruct((N_NODES, EMB_DIM), jnp.float32)]
    if not last:
        out.append(jax.ShapeDtypeStruct((N_NODES, EMB_DIM), jnp.float32))
    return out


def _layer_scratch(last):
    scratch = [
        pltpu.VMEM((B,), jnp.int32),
        pltpu.VMEM((B,), jnp.int32),
        pltpu.VMEM((B,), jnp.int32),
        pltpu.VMEM((B,), jnp.int32),
        pltpu.VMEM((B,), jnp.int32),
        pltpu.VMEM((B,), jnp.int32),
        pltpu.VMEM((B, EMB_DIM), jnp.float32),
        pltpu.VMEM((B, EMB_DIM), jnp.float32),
        pltpu.VMEM((Z_CH, EMB_DIM), jnp.float32),
        pltpu.VMEM((CH, EMB_DIM), jnp.float32),
        pltpu.VMEM((CH, EMB_DIM), jnp.float32),
        pltpu.VMEM((CH, EMB_DIM), jnp.float32),
    ]
    if not last:
        scratch.append(pltpu.VMEM((CH, EMB_DIM), jnp.float32))
    scratch += [
        pltpu.VMEM((CH,), jnp.float32),
        pltpu.VMEM_SHARED((ACC_ROWS, EMB_DIM), jnp.float32),
        pltpu.SemaphoreType.DMA,
        pltpu.SemaphoreType.DMA,
    ]
    return scratch


_mid_layer = pl.kernel(
    functools.partial(_layer_kernel_body, False),
    out_type=_layer_out_type(False),
    mesh=_MESH,
    compiler_params=pltpu.CompilerParams(use_tc_tiling_on_sc=False),
    scratch_types=_layer_scratch(False),
)

_last_layer = pl.kernel(
    functools.partial(_layer_kernel_body, True),
    out_type=_layer_out_type(True),
    mesh=_MESH,
    compiler_params=pltpu.CompilerParams(use_tc_tiling_on_sc=False),
    scratch_types=_layer_scratch(True),
)


def kernel(edge_index, edge_attrs, table):
    del edge_attrs
    pad = E_PAD - N_EDGES
    from_p = jnp.concatenate(
        [edge_index[0], jnp.zeros((pad,), jnp.int32)])
    to_p = jnp.concatenate(
        [edge_index[1], jnp.full((pad,), N_NODES, jnp.int32)])

    dinv, y = _deg_kernel(to_p, table)
    s = table
    for layer in range(N_LAYERS):
        if layer == N_LAYERS - 1:
            (s,) = _last_layer(from_p, to_p, y, s, dinv)
        else:
            s, y = _mid_layer(from_p, to_p, y, s, dinv)
    return (table, s)


# speedup vs baseline: 9.8709x; 1.1644x over previous
import functools

import jax
import jax.numpy as jnp
from jax import lax
from jax.experimental import pallas as pl
from jax.experimental.pallas import tpu as pltpu
from jax.experimental.pallas import tpu_sc as plsc

N_NODES = 50000
EMB_DIM = 64
N_EDGES = 800000
N_LAYERS = 3

NC = 2
NS = 16
LANES = 16

HALF = N_NODES // NC
ACC_ROWS = 25088
ROWS_PER_TILE = ACC_ROWS // NS
DUMMY = HALF

B = 128
CHUNKS = -(-N_EDGES // (NS * B))
CHUNKS += CHUNKS % 2
E_PAD = NS * B * CHUNKS
EDGES_PER_TILE = B * CHUNKS

CH = 32
Z_CH = 32
SCALE_CHUNKS = -(-HALF // (NS * CH))
LAST_START = HALF - CH

_MESH = plsc.VectorSubcoreMesh(core_axis_name="c", subcore_axis_name="s")


def _lane_bcast(vec, lane):
    idx = jnp.full((LANES,), lane, jnp.int32)
    return jnp.take_along_axis(vec, idx, axis=0)


def _rsqrt16(d):
    i = lax.bitcast_convert_type(d, jnp.int32)
    i = 0x5F3759DF - jnp.right_shift(i, 1)
    y = lax.bitcast_convert_type(i, jnp.float32)
    for _ in range(3):
        y = y * (1.5 - 0.5 * d * y * y)
    return jnp.where(d >= 0.5, y, 0.0)


def _local_idx(to_v, idx_v, base):
    for g in range(B // LANES):
        t = to_v[pl.ds(g * LANES, LANES)]
        local = t - base
        ok = jnp.logical_and(local >= 0, local < HALF)
        idx_v[pl.ds(g * LANES, LANES)] = jnp.where(ok, local, DUMMY)


def _scale_chunk_start(j, s):
    g = j * NS + s
    return lax.min(g * CH, LAST_START)


GI = 8
SEG = ROWS_PER_TILE


def _deg_kernel_body(to_h, table_h, dinv_h, y0_h,
                     tob_big, hist_v, tmp_seg, seg_deg, dinvbuf, tbuf, ybuf,
                     stage_s):
    c = lax.axis_index("c")
    s = lax.axis_index("s")
    base = c * HALF
    zeros16 = jnp.zeros((LANES,), jnp.float32)
    ones16 = jnp.ones((LANES,), jnp.float32)

    def zfill(i, _):
        hist_v[pl.ds(i * LANES, LANES)] = zeros16
        return 0

    lax.fori_loop(0, ACC_ROWS // LANES, zfill, 0)

    def block_step(nb, _):
        ebase = (s * CHUNKS + nb * GI) * B
        pltpu.sync_copy(to_h.at[pl.ds(ebase, GI * B)], tob_big)
        for q in range(GI * B // LANES):
            t = tob_big[pl.ds(q * LANES, LANES)]
            local = t - base
            ok = jnp.logical_and(local >= 0, local < HALF)
            idx = jnp.where(ok, local, DUMMY)
            plsc.addupdate_scatter(hist_v, [idx], ones16)
        return 0

    lax.fori_loop(0, CHUNKS // GI, block_step, 0)

    pltpu.sync_copy(hist_v, stage_s.at[s])
    plsc.subcore_barrier()

    seg0 = s * SEG
    pltpu.sync_copy(stage_s.at[0, pl.ds(seg0, SEG)], seg_deg)
    for t in range(1, NS):
        pltpu.sync_copy(stage_s.at[t, pl.ds(seg0, SEG)], tmp_seg)

        def acc_step(i, _, t=t):
            sl = pl.ds(i * LANES, LANES)
            seg_deg[sl] = seg_deg[sl] + tmp_seg[sl]
            return 0

        lax.fori_loop(0, SEG // LANES, acc_step, 0)

    def scale_step(k, _):
        start_l = lax.min(seg0 + k * CH, HALF - CH)
        off = start_l - seg0
        gstart = base + start_l
        for g in range(CH // LANES):
            d = seg_deg[pl.ds(off + g * LANES, LANES)]
            dinvbuf[pl.ds(g * LANES, LANES)] = _rsqrt16(d)
        pltpu.sync_copy(dinvbuf, dinv_h.at[pl.ds(gstart, CH)])
        pltpu.sync_copy(table_h.at[pl.ds(gstart, CH)], tbuf)

        def group(g, _):
            dvec = dinvbuf[pl.ds(g * LANES, LANES)]
            for r16 in range(LANES):
                d = _lane_bcast(dvec, r16)
                r = g * LANES + r16
                for k2 in range(EMB_DIM // LANES):
                    sl = pl.ds(k2 * LANES, LANES)
                    ybuf[r, sl] = tbuf[r, sl] * d
            return 0

        lax.fori_loop(0, CH // LANES, group, 0)
        pltpu.sync_copy(ybuf, y0_h.at[pl.ds(gstart, CH)])
        return 0

    lax.fori_loop(0, SEG // CH, scale_step, 0)


def _layer_kernel_body(last, from_h, to_h, y_h, s_h, dinv_h, *refs):
    if last:
        (s_out_h, from_v0, from_v1, to_v0, to_v1, idx_v0, idx_v1,
         rows_v0, rows_v1, zrow,
         abuf, sbuf, obuf, dinvbuf, acc_s, gsem0, gsem1) = refs
        y_out_h = None
        ybuf = None
    else:
        (s_out_h, y_out_h, from_v0, from_v1, to_v0, to_v1, idx_v0, idx_v1,
         rows_v0, rows_v1, zrow,
         abuf, sbuf, obuf, ybuf, dinvbuf, acc_s, gsem0, gsem1) = refs
    slots_from = (from_v0, from_v1)
    slots_to = (to_v0, to_v1)
    slots_idx = (idx_v0, idx_v1)
    slots_rows = (rows_v0, rows_v1)
    slots_gsem = (gsem0, gsem1)

    c = lax.axis_index("c")
    s = lax.axis_index("s")
    base = c * HALF

    zeros16 = jnp.zeros((LANES,), jnp.float32)

    def zfill(r, _):
        for k in range(EMB_DIM // LANES):
            zrow[r, pl.ds(k * LANES, LANES)] = zeros16
        return 0

    lax.fori_loop(0, Z_CH, zfill, 0)

    def zcopy(j, _):
        pltpu.sync_copy(zrow,
                        acc_s.at[pl.ds(s * ROWS_PER_TILE + j * Z_CH, Z_CH)])
        return 0

    lax.fori_loop(0, ROWS_PER_TILE // Z_CH, zcopy, 0)
    plsc.subcore_barrier()

    def prep(j, slot):
        ebase = (s * CHUNKS + j) * B
        pltpu.sync_copy(from_h.at[pl.ds(ebase, B)], slots_from[slot])
        pltpu.sync_copy(to_h.at[pl.ds(ebase, B)], slots_to[slot])
        _local_idx(slots_to[slot], slots_idx[slot], base)
        pltpu.async_copy(
            y_h.at[slots_from[slot]], slots_rows[slot], slots_gsem[slot])

    prep(0, 0)

    def outer(i, _):
        for b in (0, 1):
            j = i * 2 + b
            nslot = 1 - b
            if b == 0:
                prep(j + 1, nslot)
            else:
                @pl.when(j + 1 < CHUNKS)
                def _():
                    prep(j + 1, nslot)
            pltpu.make_async_copy(
                y_h.at[slots_from[b]], slots_rows[b], slots_gsem[b]).wait()
            pltpu.sync_copy(slots_rows[b], acc_s.at[slots_idx[b]], add=True)
        return 0

    lax.fori_loop(0, CHUNKS // 2, outer, 0)
    plsc.subcore_barrier()

    def scale_step(j, _):
        start_l = _scale_chunk_start(j, s)
        gstart = base + start_l
        pltpu.sync_copy(acc_s.at[pl.ds(start_l, CH)], abuf)
        pltpu.sync_copy(s_h.at[pl.ds(gstart, CH)], sbuf)
        pltpu.sync_copy(dinv_h.at[pl.ds(gstart, CH)], dinvbuf)

        def group(g, _):
            dvec = dinvbuf[pl.ds(g * LANES, LANES)]
            for r16 in range(LANES):
                d = _lane_bcast(dvec, r16)
                r = g * LANES + r16
                for k in range(EMB_DIM // LANES):
                    sl = pl.ds(k * LANES, LANES)
                    e = abuf[r, sl] * d
                    sv = sbuf[r, sl] + e
                    if last:
                        sv = sv * 0.25
                    obuf[r, sl] = sv
                    if not last:
                        ybuf[r, sl] = e * d
            return 0

        lax.fori_loop(0, CH // LANES, group, 0)
        pltpu.sync_copy(obuf, s_out_h.at[pl.ds(gstart, CH)])
        if not last:
            pltpu.sync_copy(ybuf, y_out_h.at[pl.ds(gstart, CH)])
        return 0

    lax.fori_loop(0, SCALE_CHUNKS, scale_step, 0)


_deg_kernel = pl.kernel(
    _deg_kernel_body,
    out_type=[
        jax.ShapeDtypeStruct((N_NODES,), jnp.float32),
        jax.ShapeDtypeStruct((N_NODES, EMB_DIM), jnp.float32),
    ],
    mesh=_MESH,
    compiler_params=pltpu.CompilerParams(
        use_tc_tiling_on_sc=False, needs_layout_passes=False),
    scratch_types=[
        pltpu.VMEM((GI * B,), jnp.int32),
        pltpu.VMEM((ACC_ROWS,), jnp.float32),
        pltpu.VMEM((SEG,), jnp.float32),
        pltpu.VMEM((SEG,), jnp.float32),
        pltpu.VMEM((CH,), jnp.float32),
        pltpu.VMEM((CH, EMB_DIM), jnp.float32),
        pltpu.VMEM((CH, EMB_DIM), jnp.float32),
        pltpu.VMEM_SHARED((NS, ACC_ROWS), jnp.float32),
    ],
)


def _layer_out_type(last):
    out = [jax.ShapeDtypeStruct((N_NODES, EMB_DIM), jnp.float32)]
    if not last:
        out.append(jax.ShapeDtypeStruct((N_NODES, EMB_DIM), jnp.float32))
    return out


def _layer_scratch(last):
    scratch = [
        pltpu.VMEM((B,), jnp.int32),
        pltpu.VMEM((B,), jnp.int32),
        pltpu.VMEM((B,), jnp.int32),
        pltpu.VMEM((B,), jnp.int32),
        pltpu.VMEM((B,), jnp.int32),
        pltpu.VMEM((B,), jnp.int32),
        pltpu.VMEM((B, EMB_DIM), jnp.float32),
        pltpu.VMEM((B, EMB_DIM), jnp.float32),
        pltpu.VMEM((Z_CH, EMB_DIM), jnp.float32),
        pltpu.VMEM((CH, EMB_DIM), jnp.float32),
        pltpu.VMEM((CH, EMB_DIM), jnp.float32),
        pltpu.VMEM((CH, EMB_DIM), jnp.float32),
    ]
    if not last:
        scratch.append(pltpu.VMEM((CH, EMB_DIM), jnp.float32))
    scratch += [
        pltpu.VMEM((CH,), jnp.float32),
        pltpu.VMEM_SHARED((ACC_ROWS, EMB_DIM), jnp.float32),
        pltpu.SemaphoreType.DMA,
        pltpu.SemaphoreType.DMA,
    ]
    return scratch


_mid_layer = pl.kernel(
    functools.partial(_layer_kernel_body, False),
    out_type=_layer_out_type(False),
    mesh=_MESH,
    compiler_params=pltpu.CompilerParams(
        use_tc_tiling_on_sc=False, needs_layout_passes=False),
    scratch_types=_layer_scratch(False),
)

_last_layer = pl.kernel(
    functools.partial(_layer_kernel_body, True),
    out_type=_layer_out_type(True),
    mesh=_MESH,
    compiler_params=pltpu.CompilerParams(
        use_tc_tiling_on_sc=False, needs_layout_passes=False),
    scratch_types=_layer_scratch(True),
)


def kernel(edge_index, edge_attrs, table):
    del edge_attrs
    pad = E_PAD - N_EDGES
    from_p = jnp.concatenate(
        [edge_index[0], jnp.zeros((pad,), jnp.int32)])
    to_p = jnp.concatenate(
        [edge_index[1], jnp.full((pad,), N_NODES, jnp.int32)])

    dinv, y = _deg_kernel(to_p, table)
    s = table
    for layer in range(N_LAYERS):
        if layer == N_LAYERS - 1:
            (s,) = _last_layer(from_p, to_p, y, s, dinv)
        else:
            s, y = _mid_layer(from_p, to_p, y, s, dinv)
    return (table, s)
